# baseline (speedup 1.0000x reference)
import jax
import jax.numpy as jnp
from jax.experimental import pallas as pl
from jax.experimental.pallas import tpu as pltpu

_NCHUNKS = 8


def _copy_manual(z_ref, o_ref, buf, rsem, wsem):
    rows = z_ref.shape[0]
    chunk = rows // _NCHUNKS
    reads = [
        pltpu.make_async_copy(
            z_ref.at[pl.ds(i * chunk, chunk), :],
            buf.at[pl.ds(i * chunk, chunk), :],
            rsem.at[i],
        )
        for i in range(_NCHUNKS)
    ]
    writes = [
        pltpu.make_async_copy(
            buf.at[pl.ds(i * chunk, chunk), :],
            o_ref.at[pl.ds(i * chunk, chunk), :],
            wsem.at[i],
        )
        for i in range(_NCHUNKS)
    ]
    for r in reads:
        r.start()
    for i in range(_NCHUNKS):
        reads[i].wait()
        writes[i].start()
    for w in writes:
        w.wait()


def kernel(z_base_global, domain_ids):
    del domain_ids
    rows, cols = z_base_global.shape
    return pl.pallas_call(
        _copy_manual,
        in_specs=[pl.BlockSpec(memory_space=pl.ANY)],
        out_specs=pl.BlockSpec(memory_space=pl.ANY),
        out_shape=jax.ShapeDtypeStruct((rows, cols), z_base_global.dtype),
        scratch_shapes=[
            pltpu.VMEM((rows, cols), z_base_global.dtype),
            pltpu.SemaphoreType.DMA((_NCHUNKS,)),
            pltpu.SemaphoreType.DMA((_NCHUNKS,)),
        ],
        compiler_params=pltpu.CompilerParams(
            vmem_limit_bytes=128 * 1024 * 1024,
        ),
    )(z_base_global)


# speedup vs baseline: 1.0046x; 1.0046x over previous
import jax
import jax.numpy as jnp
from jax.experimental import pallas as pl
from jax.experimental.pallas import tpu as pltpu

_BLOCK_ROWS = 4096


def _copy_block(z_ref, o_ref):
    o_ref[...] = z_ref[...]


def kernel(z_base_global, domain_ids):
    del domain_ids
    rows, cols = z_base_global.shape
    grid = (rows // _BLOCK_ROWS,)
    return pl.pallas_call(
        _copy_block,
        grid=grid,
        in_specs=[pl.BlockSpec((_BLOCK_ROWS, cols), lambda i: (i, 0))],
        out_specs=pl.BlockSpec((_BLOCK_ROWS, cols), lambda i: (i, 0)),
        out_shape=jax.ShapeDtypeStruct((rows, cols), z_base_global.dtype),
        compiler_params=pltpu.CompilerParams(
            dimension_semantics=("arbitrary",),
        ),
    )(z_base_global)
